# Initial kernel scaffold; baseline (speedup 1.0000x reference)
import jax
import jax.numpy as jnp
from jax.experimental import pallas as pl


def kernel(x, edge_index, edge_weight, W_z, b_z, W_r, b_r, W_h, b_h, W_lin, b_lin):
    raise NotImplementedError("write your pallas kernel here")


# speedup vs baseline: 63.6987x; 63.6987x over previous
import functools

import jax
import jax.numpy as jnp
from jax import lax
from jax.experimental import pallas as pl
from jax.experimental.pallas import tpu as pltpu
from jax.experimental.pallas import tpu_sc as plsc

_N = 512
_OC = 256
_E = 32768
_NC = 2
_NS = 16
_NW = _NC * _NS
_EPW = _E // _NW
_CHUNK = 128
_NCHUNK = _EPW // _CHUNK
_PLANE = _N * _N
_SHARED = 2 * _PLANE
_STRIPE = _SHARED // _NS


def _sc_densify_body(rows_hbm, cols_hbm, w_hbm, z_hbm, out_hbm,
                     rv, cv, wv, idx, shared):
    c = lax.axis_index("c")
    s = lax.axis_index("s")
    wid = c * _NS + s
    base = pl.multiple_of(wid * _EPW, 8)
    stripe = pl.multiple_of(s * _STRIPE, 8)
    pltpu.sync_copy(z_hbm, shared.at[pl.ds(stripe, _STRIPE)])
    pltpu.sync_copy(rows_hbm.at[pl.ds(base, _EPW)], rv)
    pltpu.sync_copy(cols_hbm.at[pl.ds(base, _EPW)], cv)
    pltpu.sync_copy(w_hbm.at[pl.ds(base, _EPW)], wv)
    for j in range(_NCHUNK):
        for k in range(_CHUNK // 16):
            o = j * _CHUNK + k * 16
            r16 = rv[pl.ds(o, 16)]
            c16 = cv[pl.ds(o, 16)]
            idx[j, pl.ds(k * 16, 16)] = r16 * _N + c16
            idx[_NCHUNK + j, pl.ds(k * 16, 16)] = c16 * _N + r16 + _PLANE
    plsc.subcore_barrier()
    for j in range(_NCHUNK):
        pltpu.sync_copy(wv.at[pl.ds(j * _CHUNK, _CHUNK)],
                        shared.at[idx.at[j]], add=True)
        pltpu.sync_copy(wv.at[pl.ds(j * _CHUNK, _CHUNK)],
                        shared.at[idx.at[_NCHUNK + j]], add=True)
    plsc.subcore_barrier()
    pltpu.sync_copy(shared.at[pl.ds(stripe, _STRIPE)], out_hbm.at[c, s])


def _sc_densify(rows, cols, w, zeros):
    mesh = plsc.VectorSubcoreMesh(core_axis_name="c", subcore_axis_name="s")
    f = functools.partial(
        pl.kernel,
        mesh=mesh,
        out_type=jax.ShapeDtypeStruct((_NC, _NS, _STRIPE), jnp.float32),
        scratch_types=[
            pltpu.VMEM((_EPW,), jnp.int32),
            pltpu.VMEM((_EPW,), jnp.int32),
            pltpu.VMEM((_EPW,), jnp.float32),
            pltpu.VMEM((2 * _NCHUNK, _CHUNK), jnp.int32),
            pltpu.VMEM_SHARED((_SHARED,), jnp.float32),
        ],
    )(_sc_densify_body)
    return f(rows, cols, w, zeros)


def _tc_body(adj_ref, x_ref, w_ref, b_ref, wl_ref, wr_ref, blin_ref, out_ref):
    A = adj_ref[0, 0] + adj_ref[1, 0]
    AT = adj_ref[0, 1] + adj_ref[1, 1]
    deg_out = jnp.sum(AT, axis=0, keepdims=True)
    deg_in = jnp.sum(A, axis=0, keepdims=True)
    ro = jnp.where(deg_out > 0.0, 1.0 / deg_out, 0.0)
    ri = jnp.where(deg_in > 0.0, 1.0 / deg_in, 0.0)
    Mo = AT * ro
    Mi = A * ri

    mm = lambda a, b: lax.dot_general(
        a, b, (((1,), (0,)), ((), ())),
        preferred_element_type=jnp.float32, precision=lax.Precision.HIGHEST)
    mmt = lambda a, b: lax.dot_general(
        a, b, (((1,), (1,)), ((), ())),
        preferred_element_type=jnp.float32, precision=lax.Precision.HIGHEST)

    X = x_ref[...]
    t1o = mm(Mo, X)
    t1i = mm(Mi, X)
    t2o = 2.0 * mm(Mo, t1o) - X
    t2i = 2.0 * mm(Mi, t1i) - X
    G = (mm(X, w_ref[0] + w_ref[1]) + mm(t1o, w_ref[2]) + mm(t1i, w_ref[3])
         + mm(t2o, w_ref[4]) + mm(t2i, w_ref[5]) + b_ref[...])
    Z = jax.nn.sigmoid(G[:, :_OC])
    Ht = jnp.tanh(G[:, _OC:])
    out = jnp.maximum((1.0 - Z) * Ht, 0.0)
    a_row = mmt(wl_ref[...], out)
    b_col = mmt(out, wr_ref[...])
    out_ref[...] = b_col + a_row + blin_ref[...]


def kernel(x, edge_index, edge_weight, W_z, b_z, W_r, b_r, W_h, b_h,
           W_lin, b_lin):
    del W_r, b_r
    rows = edge_index[0].astype(jnp.int32)
    cols = edge_index[1].astype(jnp.int32)
    w = edge_weight.astype(jnp.float32)
    zeros = jnp.zeros((_STRIPE,), jnp.float32)
    planes = _sc_densify(rows, cols, w, zeros).reshape(_NC, 2, _N, _N)

    Wstack = jnp.stack([
        jnp.concatenate([W_z[0, 0, :_N], W_h[0, 0, :_N]], axis=1),
        jnp.concatenate([W_z[1, 0, :_N], W_h[1, 0, :_N]], axis=1),
        jnp.concatenate([W_z[0, 1, :_N], W_h[0, 1, :_N]], axis=1),
        jnp.concatenate([W_z[1, 1, :_N], W_h[1, 1, :_N]], axis=1),
        jnp.concatenate([W_z[0, 2, :_N], W_h[0, 2, :_N]], axis=1),
        jnp.concatenate([W_z[1, 2, :_N], W_h[1, 2, :_N]], axis=1),
    ])
    bcat = jnp.concatenate([b_z, b_h])[None, :]
    wl = W_lin[:, :_OC]
    wr = W_lin[:, _OC:]
    blin = b_lin.reshape(1, 1)

    res = pl.pallas_call(
        _tc_body,
        out_shape=jax.ShapeDtypeStruct((_N, _N), jnp.float32),
    )(planes, x, Wstack, bcat, wl, wr, blin)
    return res.reshape(_N * _N, 1)


# speedup vs baseline: 70.6160x; 1.1086x over previous
import functools

import jax
import jax.numpy as jnp
from jax import lax
from jax.experimental import pallas as pl
from jax.experimental.pallas import tpu as pltpu
from jax.experimental.pallas import tpu_sc as plsc

_N = 512
_OC = 256
_E = 32768
_NC = 2
_NS = 16
_EPW = _E // _NS
_CHUNK = 128
_NCHUNK = _EPW // _CHUNK
_PLANE = _N * _N
_STRIPE = _PLANE // _NS


def _sc_densify_body(rows_hbm, cols_hbm, w_hbm, z_hbm, out_hbm,
                     rv, cv, wv, idx, shared):
    c = lax.axis_index("c")
    s = lax.axis_index("s")
    base = pl.multiple_of(s * _EPW, 8)
    stripe = pl.multiple_of(s * _STRIPE, 8)
    pltpu.sync_copy(z_hbm, shared.at[pl.ds(stripe, _STRIPE)])
    pltpu.sync_copy(rows_hbm.at[pl.ds(base, _EPW)], rv)
    pltpu.sync_copy(cols_hbm.at[pl.ds(base, _EPW)], cv)
    pltpu.sync_copy(w_hbm.at[pl.ds(base, _EPW)], wv)
    is_a = c == 0
    for j in range(_NCHUNK):
        for k in range(_CHUNK // 16):
            o = j * _CHUNK + k * 16
            r16 = rv[pl.ds(o, 16)]
            c16 = cv[pl.ds(o, 16)]
            idx[j, pl.ds(k * 16, 16)] = jnp.where(
                is_a, r16 * _N + c16, c16 * _N + r16)
    plsc.subcore_barrier()
    for j in range(_NCHUNK):
        pltpu.sync_copy(wv.at[pl.ds(j * _CHUNK, _CHUNK)],
                        shared.at[idx.at[j]], add=True)
    plsc.subcore_barrier()
    pltpu.sync_copy(shared.at[pl.ds(stripe, _STRIPE)], out_hbm.at[c, s])


def _sc_densify(rows, cols, w, zeros):
    mesh = plsc.VectorSubcoreMesh(core_axis_name="c", subcore_axis_name="s")
    f = functools.partial(
        pl.kernel,
        mesh=mesh,
        out_type=jax.ShapeDtypeStruct((_NC, _NS, _STRIPE), jnp.float32),
        scratch_types=[
            pltpu.VMEM((_EPW,), jnp.int32),
            pltpu.VMEM((_EPW,), jnp.int32),
            pltpu.VMEM((_EPW,), jnp.float32),
            pltpu.VMEM((_NCHUNK, _CHUNK), jnp.int32),
            pltpu.VMEM_SHARED((_PLANE,), jnp.float32),
        ],
    )(_sc_densify_body)
    return f(rows, cols, w, zeros)


def _tc_body(adj_ref, x_ref, wz_ref, wh_ref, b_ref, wl_ref, wr_ref, blin_ref,
             out_ref):
    A = adj_ref[0]
    AT = adj_ref[1]
    deg_out = jnp.sum(AT, axis=0, keepdims=True)
    deg_in = jnp.sum(A, axis=0, keepdims=True)
    ro = jnp.where(deg_out > 0.0, 1.0 / deg_out, 0.0)
    ri = jnp.where(deg_in > 0.0, 1.0 / deg_in, 0.0)
    Mo = AT * ro
    Mi = A * ri

    mm = lambda a, b: lax.dot_general(
        a, b, (((1,), (0,)), ((), ())),
        preferred_element_type=jnp.float32, precision=lax.Precision.HIGHEST)
    mmt = lambda a, b: lax.dot_general(
        a, b, (((1,), (1,)), ((), ())),
        preferred_element_type=jnp.float32, precision=lax.Precision.HIGHEST)

    X = x_ref[...]
    t1o = mm(Mo, X)
    t1i = mm(Mi, X)
    t2o = 2.0 * mm(Mo, t1o) - X
    t2i = 2.0 * mm(Mi, t1i) - X
    def gate(w_ref, off):
        return (mm(X, w_ref[0, 0, :_N, :] + w_ref[1, 0, :_N, :])
                + mm(t1o, w_ref[0, 1, :_N, :]) + mm(t1i, w_ref[1, 1, :_N, :])
                + mm(t2o, w_ref[0, 2, :_N, :]) + mm(t2i, w_ref[1, 2, :_N, :])
                + b_ref[:, off:off + _OC])
    Z = jax.nn.sigmoid(gate(wz_ref, 0))
    Ht = jnp.tanh(gate(wh_ref, _OC))
    out = jnp.maximum((1.0 - Z) * Ht, 0.0)
    a_row = mmt(wl_ref[...], out)
    b_col = mmt(out, wr_ref[...])
    out_ref[...] = b_col + a_row + blin_ref[...]


def kernel(x, edge_index, edge_weight, W_z, b_z, W_r, b_r, W_h, b_h,
           W_lin, b_lin):
    del W_r, b_r
    rows = edge_index[0].astype(jnp.int32)
    cols = edge_index[1].astype(jnp.int32)
    w = edge_weight.astype(jnp.float32)
    zeros = jnp.zeros((_STRIPE,), jnp.float32)
    planes = _sc_densify(rows, cols, w, zeros).reshape(_NC, _N, _N)

    bcat = jnp.concatenate([b_z, b_h])[None, :]
    wl = W_lin[:, :_OC]
    wr = W_lin[:, _OC:]
    blin = b_lin.reshape(1, 1)

    res = pl.pallas_call(
        _tc_body,
        out_shape=jax.ShapeDtypeStruct((_N, _N), jnp.float32),
    )(planes, x, W_z, W_h, bcat, wl, wr, blin)
    return res.reshape(_N * _N, 1)


# speedup vs baseline: 92.9860x; 1.3168x over previous
import functools

import jax
import jax.numpy as jnp
from jax import lax
from jax.experimental import pallas as pl
from jax.experimental.pallas import tpu as pltpu
from jax.experimental.pallas import tpu_sc as plsc

_N = 512
_OC = 256
_E = 32768
_NC = 2
_NS = 16
_EPW = _E // _NS
_CHUNK = 128
_NCHUNK = _EPW // _CHUNK
_PLANE = _N * _N
_STRIPE = _PLANE // _NS


def _sc_densify_body(rows_hbm, cols_hbm, w_hbm, z_hbm, out_hbm,
                     rv, cv, wv, idx, shared):
    c = lax.axis_index("c")
    s = lax.axis_index("s")
    base = pl.multiple_of(s * _EPW, 8)
    stripe = pl.multiple_of(s * _STRIPE, 8)
    pltpu.sync_copy(z_hbm, shared.at[pl.ds(stripe, _STRIPE)])
    pltpu.sync_copy(rows_hbm.at[pl.ds(base, _EPW)], rv)
    pltpu.sync_copy(cols_hbm.at[pl.ds(base, _EPW)], cv)
    pltpu.sync_copy(w_hbm.at[pl.ds(base, _EPW)], wv)
    is_a = c == 0
    for j in range(_NCHUNK):
        for k in range(_CHUNK // 16):
            o = j * _CHUNK + k * 16
            r16 = rv[pl.ds(o, 16)]
            c16 = cv[pl.ds(o, 16)]
            idx[j, pl.ds(k * 16, 16)] = jnp.where(
                is_a, r16 * _N + c16, c16 * _N + r16)
    plsc.subcore_barrier()
    for j in range(_NCHUNK):
        pltpu.sync_copy(wv.at[pl.ds(j * _CHUNK, _CHUNK)],
                        shared.at[idx.at[j]], add=True)
    plsc.subcore_barrier()
    pltpu.sync_copy(shared.at[pl.ds(stripe, _STRIPE)], out_hbm.at[c, s])


def _sc_densify(rows, cols, w, zeros):
    mesh = plsc.VectorSubcoreMesh(core_axis_name="c", subcore_axis_name="s")
    f = functools.partial(
        pl.kernel,
        mesh=mesh,
        out_type=jax.ShapeDtypeStruct((_NC, _NS, _STRIPE), jnp.float32),
        scratch_types=[
            pltpu.VMEM((_EPW,), jnp.int32),
            pltpu.VMEM((_EPW,), jnp.int32),
            pltpu.VMEM((_EPW,), jnp.float32),
            pltpu.VMEM((_NCHUNK, _CHUNK), jnp.int32),
            pltpu.VMEM_SHARED((_PLANE,), jnp.float32),
        ],
    )(_sc_densify_body)
    return f(rows, cols, w, zeros)


def _tc_body(adj_ref, x_ref, wz_ref, wh_ref, b_ref, wl_ref, wr_ref, blin_ref,
             out_ref):
    A = adj_ref[0]
    AT = adj_ref[1]
    deg_out = jnp.sum(AT, axis=0, keepdims=True)
    deg_in = jnp.sum(A, axis=0, keepdims=True)
    ro = jnp.where(deg_out > 0.0, 1.0 / deg_out, 0.0)
    ri = jnp.where(deg_in > 0.0, 1.0 / deg_in, 0.0)
    Mo = AT * ro
    Mi = A * ri

    mm = lambda a, b: lax.dot_general(
        a, b, (((1,), (0,)), ((), ())),
        preferred_element_type=jnp.float32, precision=lax.Precision.DEFAULT)
    mmt = lambda a, b: lax.dot_general(
        a, b, (((1,), (1,)), ((), ())),
        preferred_element_type=jnp.float32, precision=lax.Precision.DEFAULT)

    X = x_ref[...]
    t1o = mm(Mo, X)
    t1i = mm(Mi, X)
    t2o = 2.0 * mm(Mo, t1o) - X
    t2i = 2.0 * mm(Mi, t1i) - X
    def gate(w_ref, off):
        return (mm(X, w_ref[0, 0, :_N, :] + w_ref[1, 0, :_N, :])
                + mm(t1o, w_ref[0, 1, :_N, :]) + mm(t1i, w_ref[1, 1, :_N, :])
                + mm(t2o, w_ref[0, 2, :_N, :]) + mm(t2i, w_ref[1, 2, :_N, :])
                + b_ref[:, off:off + _OC])
    Z = jax.nn.sigmoid(gate(wz_ref, 0))
    Ht = jnp.tanh(gate(wh_ref, _OC))
    out = jnp.maximum((1.0 - Z) * Ht, 0.0)
    a_row = mmt(wl_ref[...], out)
    b_col = mmt(out, wr_ref[...])
    out_ref[...] = b_col + a_row + blin_ref[...]


def kernel(x, edge_index, edge_weight, W_z, b_z, W_r, b_r, W_h, b_h,
           W_lin, b_lin):
    del W_r, b_r
    rows = edge_index[0].astype(jnp.int32)
    cols = edge_index[1].astype(jnp.int32)
    w = edge_weight.astype(jnp.float32)
    zeros = jnp.zeros((_STRIPE,), jnp.float32)
    planes = _sc_densify(rows, cols, w, zeros).reshape(_NC, _N, _N)

    bcat = jnp.concatenate([b_z, b_h])[None, :]
    wl = W_lin[:, :_OC]
    wr = W_lin[:, _OC:]
    blin = b_lin.reshape(1, 1)

    res = pl.pallas_call(
        _tc_body,
        out_shape=jax.ShapeDtypeStruct((_N, _N), jnp.float32),
    )(planes, x, W_z, W_h, bcat, wl, wr, blin)
    return res.reshape(_N * _N, 1)
